# Initial kernel scaffold; baseline (speedup 1.0000x reference)
import jax
import jax.numpy as jnp
from jax.experimental import pallas as pl


def kernel(obj_vecs, rela_vecs, rela_masks, W, b, edges):
    raise NotImplementedError("write your pallas kernel here")


# speedup vs baseline: 2.6530x; 2.6530x over previous
import functools

import jax
import jax.numpy as jnp
from jax import lax
from jax.experimental import pallas as pl
from jax.experimental.pallas import tpu as pltpu
from jax.experimental.pallas import tpu_sc as plsc

_L = 16


def _proj_body(x_ref, ws_ref, wo_ref, ps_ref, po_ref):
    x = x_ref[...]
    ps_ref[...] = jnp.dot(x, ws_ref[...], preferred_element_type=jnp.float32)
    po_ref[...] = jnp.dot(x, wo_ref[...], preferred_element_type=jnp.float32)


def _project_objects(obj2d, ws_t, wo_t):
    n, d = obj2d.shape
    blk = 1000
    grid = n // blk
    return pl.pallas_call(
        _proj_body,
        grid=(grid,),
        in_specs=[
            pl.BlockSpec((blk, d), lambda i: (i, 0)),
            pl.BlockSpec((d, d), lambda i: (0, 0)),
            pl.BlockSpec((d, d), lambda i: (0, 0)),
        ],
        out_specs=[
            pl.BlockSpec((blk, d), lambda i: (i, 0)),
            pl.BlockSpec((blk, d), lambda i: (i, 0)),
        ],
        out_shape=[
            jax.ShapeDtypeStruct((n, d), jnp.float32),
            jax.ShapeDtypeStruct((n, d), jnp.float32),
        ],
    )(obj2d, ws_t, wo_t)


def _make_gather_sum(nw, nch, k, d):
    mesh = plsc.VectorSubcoreMesh(core_axis_name="c", subcore_axis_name="s")

    @functools.partial(
        pl.kernel,
        mesh=mesh,
        out_type=jax.ShapeDtypeStruct((nw, nch, k, d), jnp.float32),
        scratch_types=[
            pltpu.VMEM((2, k), jnp.int32),
            pltpu.VMEM((2, k), jnp.int32),
            pltpu.VMEM((2, k, d), jnp.float32),
            pltpu.VMEM((2, k, d), jnp.float32),
            pltpu.VMEM((2, k, d), jnp.float32),
            pltpu.SemaphoreType.DMA,
            pltpu.SemaphoreType.DMA,
            pltpu.SemaphoreType.DMA,
            pltpu.SemaphoreType.DMA,
            pltpu.SemaphoreType.DMA,
            pltpu.SemaphoreType.DMA,
            pltpu.SemaphoreType.DMA,
            pltpu.SemaphoreType.DMA,
        ],
    )
    def gather_sum(sidx_hbm, oidx_hbm, ps_hbm, po_hbm, out_hbm,
                   idx_s, idx_o, rows_s, rows_o, obuf,
                   isem_s, isem_o, gs0, gs1, go0, go1, os0, os1):
        wid = lax.axis_index("s") * 2 + lax.axis_index("c")
        gsem_s = (gs0, gs1)
        gsem_o = (go0, go1)
        osem = (os0, os1)

        def idx_copies(c, p):
            return (pltpu.make_async_copy(sidx_hbm.at[wid, c], idx_s.at[p],
                                          isem_s),
                    pltpu.make_async_copy(oidx_hbm.at[wid, c], idx_o.at[p],
                                          isem_o))

        def gather_copies(p):
            return (pltpu.make_async_copy(ps_hbm.at[idx_s.at[p]],
                                          rows_s.at[p], gsem_s[p]),
                    pltpu.make_async_copy(po_hbm.at[idx_o.at[p]],
                                          rows_o.at[p], gsem_o[p]))

        def out_copy(c, p):
            return pltpu.make_async_copy(obuf.at[p], out_hbm.at[wid, c],
                                         osem[p])

        def compute(p):
            @plsc.parallel_loop(0, k, unroll=4)
            def _(i):
                for j in range(d // _L):
                    sl = pl.ds(j * _L, _L)
                    obuf[p, i, sl] = rows_s[p, i, sl] + rows_o[p, i, sl]

        for cp in idx_copies(0, 0):
            cp.start()
            cp.wait()
        for cp in gather_copies(0):
            cp.start()
        for cp in idx_copies(1, 1):
            cp.start()

        def step(c, p):
            q = p ^ 1

            @pl.when(c + 1 < nch)
            def _():
                for cp in idx_copies(c + 1, q):
                    cp.wait()
            for cp in gather_copies(p):
                cp.wait()

            @pl.when(c + 1 < nch)
            def _():
                for cp in gather_copies(q):
                    cp.start()

            @pl.when(c + 2 < nch)
            def _():
                for cp in idx_copies(c + 2, p):
                    cp.start()

            @pl.when(c >= 2)
            def _():
                out_copy(c - 2, p).wait()

            compute(p)
            out_copy(c, p).start()

        def pair_body(cc, carry):
            step(2 * cc, 0)

            @pl.when(2 * cc + 1 < nch)
            def _():
                step(2 * cc + 1, 1)

            return carry

        lax.fori_loop(0, (nch + 1) // 2, pair_body, 0)

        last = nch - 1
        out_copy(last, last & 1).wait()
        out_copy(last - 1, (last - 1) & 1).wait()

    return gather_sum


def _final_body(r_ref, g_ref, w_ref, b_ref, m_ref, o_ref):
    z = jnp.dot(r_ref[...], w_ref[...], preferred_element_type=jnp.float32)
    z = z + b_ref[...] + g_ref[...]
    o_ref[...] = jnp.maximum(z, 0.0) * m_ref[...]


def _final(rela2d, gsum, wr_t, b, mask2d):
    n, d = rela2d.shape
    blk = 2000
    grid = n // blk
    return pl.pallas_call(
        _final_body,
        grid=(grid,),
        in_specs=[
            pl.BlockSpec((blk, d), lambda i: (i, 0)),
            pl.BlockSpec((blk, d), lambda i: (i, 0)),
            pl.BlockSpec((d, d), lambda i: (0, 0)),
            pl.BlockSpec((1, d), lambda i: (0, 0)),
            pl.BlockSpec((blk, 1), lambda i: (i, 0)),
        ],
        out_specs=pl.BlockSpec((blk, d), lambda i: (i, 0)),
        out_shape=jax.ShapeDtypeStruct((n, d), jnp.float32),
    )(rela2d, gsum, wr_t, b, mask2d)


def kernel(obj_vecs, rela_vecs, rela_masks, W, b, edges):
    bsz, no, d = obj_vecs.shape
    nr = rela_vecs.shape[1]
    e = bsz * nr

    obj2d = obj_vecs.reshape(-1, d)
    rela2d = rela_vecs.reshape(-1, d)
    ws_t = W[:, :d].T
    wr_t = W[:, d:2 * d].T
    wo_t = W[:, 2 * d:].T

    offs = (jnp.arange(bsz) * no).astype(edges.dtype)
    ge = (edges + offs[:, None, None]).reshape(-1, 2).astype(jnp.int32)
    nw, k = 32, 128
    nch = -(-e // (nw * k))
    pad = nw * nch * k - e
    sidx = jnp.concatenate([ge[:, 0], jnp.zeros((pad,), jnp.int32)])
    oidx = jnp.concatenate([ge[:, 1], jnp.zeros((pad,), jnp.int32)])
    sidx = sidx.reshape(nw, nch, k)
    oidx = oidx.reshape(nw, nch, k)

    proj_s, proj_o = _project_objects(obj2d, ws_t, wo_t)

    gsum = _make_gather_sum(nw, nch, k, d)(sidx, oidx, proj_s, proj_o)
    gsum = gsum.reshape(nw * nch * k, d)

    out2d = _final(rela2d, gsum, wr_t, b.reshape(1, d),
                   rela_masks.reshape(e, 1))
    return (obj_vecs, out2d.reshape(bsz, nr, d))
